# baseline (speedup 1.0000x reference)
import jax
import jax.numpy as jnp
from jax.experimental import pallas as pl
from jax.experimental.pallas import tpu as pltpu

_BM = 4096
_K = 512
_D = 64
_G = 8


def _argmax_half(ft, c, out_ref):
    norm = jnp.sqrt(jnp.sum(ft * ft, axis=0, keepdims=True))
    fnt = ft / jnp.maximum(norm, 1e-12)
    sim = jax.lax.dot_general(
        c, fnt, (((1,), (0,)), ((), ())), preferred_element_type=jnp.float32
    )

    n_groups = _K // _G
    groups = [sim[g * _G:(g + 1) * _G, :] for g in range(n_groups)]
    best_e = groups[0]
    best_v = jnp.maximum(groups[0], groups[1])
    best_p = jnp.zeros((_G, _BM), jnp.float32)
    for p in range(1, n_groups // 2):
        e = groups[2 * p]
        v = jnp.maximum(e, groups[2 * p + 1])
        take = v > best_v
        best_v = jnp.maximum(best_v, v)
        best_p = jnp.where(take, jnp.float32(p), best_p)
        best_e = jnp.where(take, e, best_e)

    best_g = best_p * 2.0 + jnp.where(best_e == best_v, 0.0, 1.0)

    m = jnp.max(best_v, axis=0, keepdims=True)
    r = jax.lax.broadcasted_iota(jnp.int32, (_G, 1), 0).astype(jnp.float32)
    idx = jnp.min(
        jnp.where(best_v == m, best_g * jnp.float32(_G) + r, jnp.float32(_K)),
        axis=0,
    )
    out_ref[...] = idx.astype(jnp.int32)


def _assign_kernel(x_ref, c_ref, oa_ref, ob_ref):
    x = x_ref[...]
    c = c_ref[...]
    _argmax_half(x[:, :_D].T, c, oa_ref)
    _argmax_half(x[:, _D:].T, c, ob_ref)


def kernel(features, centroids):
    n = features.shape[0]
    h = n // 2
    merged = jnp.concatenate([features[:h], features[h:]], axis=1)
    oa, ob = pl.pallas_call(
        _assign_kernel,
        grid=(h // _BM,),
        in_specs=[
            pl.BlockSpec((_BM, 2 * _D), lambda i: (i, 0)),
            pl.BlockSpec((_K, _D), lambda i: (0, 0)),
        ],
        out_specs=[
            pl.BlockSpec((_BM,), lambda i: (i,)),
            pl.BlockSpec((_BM,), lambda i: (i,)),
        ],
        out_shape=[
            jax.ShapeDtypeStruct((h,), jnp.int32),
            jax.ShapeDtypeStruct((h,), jnp.int32),
        ],
        compiler_params=pltpu.CompilerParams(
            dimension_semantics=("parallel",),
        ),
    )(merged, centroids)
    return jnp.concatenate([oa, ob])


# speedup vs baseline: 1.6441x; 1.6441x over previous
import jax
import jax.numpy as jnp
from jax.experimental import pallas as pl
from jax.experimental.pallas import tpu as pltpu

_BN = 16384
_K = 512
_D = 64
_G = 8


def _assign_kernel(f_ref, c_ref, out_ref):
    f = f_ref[...]
    c = c_ref[...]
    ft = f.T
    norm = jnp.sqrt(jnp.sum(ft * ft, axis=0, keepdims=True))
    fnt = ft / jnp.maximum(norm, 1e-12)
    sim = jax.lax.dot_general(
        c, fnt, (((1,), (0,)), ((), ())), preferred_element_type=jnp.float32
    )

    n_groups = _K // _G
    groups = [sim[g * _G:(g + 1) * _G, :] for g in range(n_groups)]
    best_e = groups[0]
    best_v = jnp.maximum(groups[0], groups[1])
    best_p = jnp.zeros((_G, _BN), jnp.float32)
    for p in range(1, n_groups // 2):
        e = groups[2 * p]
        v = jnp.maximum(e, groups[2 * p + 1])
        take = v > best_v
        best_v = jnp.maximum(best_v, v)
        best_p = jnp.where(take, jnp.float32(p), best_p)
        best_e = jnp.where(take, e, best_e)

    best_g = best_p * 2.0 + jnp.where(best_e == best_v, 0.0, 1.0)

    m = jnp.max(best_v, axis=0, keepdims=True)
    r = jax.lax.broadcasted_iota(jnp.int32, (_G, 1), 0).astype(jnp.float32)
    idx = jnp.min(
        jnp.where(best_v == m, best_g * jnp.float32(_G) + r, jnp.float32(_K)),
        axis=0,
    )
    out_ref[...] = idx.astype(jnp.int32)


def kernel(features, centroids):
    n = features.shape[0]
    grid = (n // _BN,)
    assignments = pl.pallas_call(
        _assign_kernel,
        grid=grid,
        in_specs=[
            pl.BlockSpec((_BN, _D), lambda i: (i, 0)),
            pl.BlockSpec((_K, _D), lambda i: (0, 0)),
        ],
        out_specs=pl.BlockSpec((_BN,), lambda i: (i,)),
        out_shape=jax.ShapeDtypeStruct((n,), jnp.int32),
        compiler_params=pltpu.CompilerParams(
            dimension_semantics=("parallel",),
            allow_input_fusion=[True, True],
        ),
    )(features, centroids)
    return assignments
